# baseline (speedup 1.0000x reference)
import jax
import jax.numpy as jnp
import numpy as np
from jax.experimental import pallas as pl
from jax.experimental.pallas import tpu as pltpu
from jax.sharding import Mesh, PartitionSpec as P

_TB = 8


def _fused_body(TB, Bs, I, H, O):
    f32 = jnp.float32
    bf16 = jnp.bfloat16

    def body(xs_ref, h0_ref, wcat_ref, bcat_ref, whh_ref, mw_ref, wouh_ref,
             out_ref, hlast_ref, hstack_ref):
        c = pl.program_id(0)

        @pl.when(c == 0)
        def _():
            hlast_ref[...] = h0_ref[...]

        x = xs_ref[...].reshape(TB * Bs, I).astype(bf16)
        z = jnp.dot(x, wcat_ref[...], preferred_element_type=f32) + bcat_ref[...]

        h = hlast_ref[...]
        hstack_ref[0:Bs, :] = h.astype(bf16)
        for i in range(TB):
            hb = h.astype(bf16)
            h = z[i * Bs:(i + 1) * Bs, :H] + jnp.dot(
                hb, whh_ref[...], preferred_element_type=f32)
            hstack_ref[(i + 1) * Bs:(i + 2) * Bs, :] = h.astype(bf16)
        hlast_ref[...] = h

        logits = (z[:, H:]
                  + jnp.dot(hstack_ref[0:TB * Bs, :], mw_ref[...],
                            preferred_element_type=f32)
                  + jnp.dot(hstack_ref[Bs:(TB + 1) * Bs, :], wouh_ref[...],
                            preferred_element_type=f32))
        mx = jnp.max(logits, axis=-1, keepdims=True)
        y = logits - mx
        lse = jnp.log(jnp.sum(jnp.exp(y), axis=-1, keepdims=True))
        out_ref[...] = (y - lse).reshape(TB, Bs, O)

    return body


def _rnn_scan(xs, h0, wcat, bcat, whh, m_w, wouh, H, O):
    T, Bs, I = xs.shape
    f32 = jnp.float32
    bf16 = jnp.bfloat16
    TB = _TB
    n_chunks = T // TB
    IO = wcat.shape[1]

    return pl.pallas_call(
        _fused_body(TB, Bs, I, H, O),
        grid=(n_chunks,),
        in_specs=[
            pl.BlockSpec((TB, Bs, I), lambda c: (c, 0, 0)),
            pl.BlockSpec((Bs, H), lambda c: (0, 0)),
            pl.BlockSpec((I, IO), lambda c: (0, 0)),
            pl.BlockSpec((1, IO), lambda c: (0, 0)),
            pl.BlockSpec((H, H), lambda c: (0, 0)),
            pl.BlockSpec((H, O), lambda c: (0, 0)),
            pl.BlockSpec((H, O), lambda c: (0, 0)),
        ],
        out_specs=[
            pl.BlockSpec((TB, Bs, O), lambda c: (c, 0, 0)),
            pl.BlockSpec((Bs, H), lambda c: (0, 0)),
        ],
        out_shape=(
            jax.ShapeDtypeStruct((T, Bs, O), f32),
            jax.ShapeDtypeStruct((Bs, H), f32),
        ),
        scratch_shapes=[pltpu.VMEM(((TB + 1) * Bs, H), bf16)],
        compiler_params=pltpu.CompilerParams(
            dimension_semantics=("arbitrary",),
        ),
    )(xs, h0, wcat, bcat, whh, m_w, wouh)


def kernel(xs, h0, wih_x, b_ih, wio_x, b_io, whh, wio_h, wou_o, wou_h, bou):
    T, B, I = xs.shape
    H = whh.shape[0]
    O = wou_o.shape[0]
    f32 = jnp.float32
    bf16 = jnp.bfloat16

    wou_f = wou_o.astype(f32)
    wfold = jnp.dot(wio_x, wou_f)
    bfold = jnp.dot(b_io, wou_f) + bou[0]
    wcat = jnp.concatenate([wih_x, wfold], axis=1).astype(bf16)
    bcat = jnp.concatenate([b_ih, bfold]).reshape(1, H + O)
    m_w = jnp.dot(wio_h.astype(f32), wou_f).astype(bf16)

    devs = jax.devices()
    n_shard = 2 if (len(devs) >= 2 and B % 2 == 0) else 1
    if n_shard == 1:
        return _rnn_scan(xs, h0, wcat, bcat, whh, m_w, wouh=wou_h, H=H, O=O)

    mesh = Mesh(np.array(devs[:n_shard]), ("b",))

    def sharded(xs_s, h0_s, wcat_s, bcat_s, whh_s, mw_s, wouh_s):
        return _rnn_scan(xs_s, h0_s, wcat_s, bcat_s, whh_s, mw_s, wouh_s,
                         H=H, O=O)

    ys, h_last = jax.shard_map(
        sharded,
        mesh=mesh,
        in_specs=(P(None, "b", None), P("b", None), P(), P(), P(), P(), P()),
        out_specs=(P(None, "b", None), P("b", None)),
        check_vma=False,
    )(xs, h0, wcat, bcat, whh, m_w, wou_h)
    return ys, h_last


# speedup vs baseline: 2.9041x; 2.9041x over previous
import jax
import jax.numpy as jnp
import numpy as np
from jax.experimental import pallas as pl
from jax.experimental.pallas import tpu as pltpu
from jax.sharding import Mesh, PartitionSpec as P

_TB = 8


def _fused_body(TB, Bs, I, H, O):
    f32 = jnp.float32
    bf16 = jnp.bfloat16

    def body(xs_ref, h0_ref, wcat_ref, bcat_ref, whh_ref, mw_ref, wouh_ref,
             out_ref, hlast_ref, hstack_ref):
        c = pl.program_id(0)

        @pl.when(c == 0)
        def _():
            hlast_ref[...] = h0_ref[...]

        x = xs_ref[...].reshape(TB * Bs, I).astype(bf16)
        z = jnp.dot(x, wcat_ref[...], preferred_element_type=f32) + bcat_ref[...]

        h = hlast_ref[...]
        hstack_ref[0:Bs, :] = h.astype(bf16)
        for i in range(TB):
            hb = h.astype(bf16)
            h = z[i * Bs:(i + 1) * Bs, :H] + jnp.dot(
                hb, whh_ref[...], preferred_element_type=f32)
            hstack_ref[(i + 1) * Bs:(i + 2) * Bs, :] = h.astype(bf16)
        hlast_ref[...] = h

        logits = (z[:, H:]
                  + jnp.dot(hstack_ref[0:TB * Bs, :], mw_ref[...],
                            preferred_element_type=f32)
                  + jnp.dot(hstack_ref[Bs:(TB + 1) * Bs, :], wouh_ref[...],
                            preferred_element_type=f32))
        mx = jnp.max(logits, axis=-1, keepdims=True)
        y = logits - mx
        lse = jnp.log(jnp.sum(jnp.exp(y), axis=-1, keepdims=True))
        out_ref[...] = (y - lse).reshape(TB, Bs, O)

    return body


def _rnn_scan(xs, h0, wcat, bcat, whh, m_w, wouh, H, O):
    T, Bs, I = xs.shape
    f32 = jnp.float32
    bf16 = jnp.bfloat16
    TB = _TB
    n_chunks = T // TB
    IO = wcat.shape[1]

    return pl.pallas_call(
        _fused_body(TB, Bs, I, H, O),
        grid=(n_chunks,),
        in_specs=[
            pl.BlockSpec((TB, Bs, I), lambda c: (c, 0, 0)),
            pl.BlockSpec((Bs, H), lambda c: (0, 0)),
            pl.BlockSpec((I, IO), lambda c: (0, 0)),
            pl.BlockSpec((1, IO), lambda c: (0, 0)),
            pl.BlockSpec((H, H), lambda c: (0, 0)),
            pl.BlockSpec((H, O), lambda c: (0, 0)),
            pl.BlockSpec((H, O), lambda c: (0, 0)),
        ],
        out_specs=[
            pl.BlockSpec((TB, Bs, O), lambda c: (c, 0, 0)),
            pl.BlockSpec((Bs, H), lambda c: (0, 0)),
        ],
        out_shape=(
            jax.ShapeDtypeStruct((T, Bs, O), f32),
            jax.ShapeDtypeStruct((Bs, H), f32),
        ),
        scratch_shapes=[pltpu.VMEM(((TB + 1) * Bs, H), bf16)],
        compiler_params=pltpu.CompilerParams(
            dimension_semantics=("arbitrary",),
        ),
    )(xs, h0, wcat, bcat, whh, m_w, wouh)


def kernel(xs, h0, wih_x, b_ih, wio_x, b_io, whh, wio_h, wou_o, wou_h, bou):
    T, B, I = xs.shape
    H = whh.shape[0]
    O = wou_o.shape[0]
    f32 = jnp.float32
    bf16 = jnp.bfloat16

    wou_f = wou_o.astype(f32)
    wfold = jnp.dot(wio_x, wou_f)
    bfold = jnp.dot(b_io, wou_f) + bou[0]
    wcat = jnp.concatenate([wih_x, wfold], axis=1).astype(bf16)
    bcat = jnp.concatenate([b_ih, bfold]).reshape(1, H + O)
    m_w = jnp.dot(wio_h.astype(f32), wou_f).astype(bf16)

    return _rnn_scan(xs, h0, wcat, bcat, whh, m_w, wouh=wou_h, H=H, O=O)
